# baseline (speedup 1.0000x reference)
import jax
import jax.numpy as jnp
from jax import lax
from jax.experimental import pallas as pl
from jax.experimental.pallas import tpu as pltpu
from jax.experimental.pallas import tpu_sc as plsc

_B = 16384
_H = 32
_E = 1000000
_R = 1000
_NW = 32
_BPW = _B // _NW
_CH = 64
_NCH = _BPW // _CH
_NL = _CH * _H // 128
_LMBDA = 0.0001


def _sc_body(h1, t1, r1i, y_in, e1f, e2f, rl1f, rl2f,
             res_out, regul_out,
             hraw, traw, rraw, hidx, tidx,
             be1h, be2h, be1t, be2t,
             rl1res, rl2res,
             drainbuf, resbuf, accbuf, sems):
    nc = 2
    wid = lax.axis_index("s") * nc + lax.axis_index("c")
    base_w = wid * _BPW

    pltpu.sync_copy(h1.at[pl.ds(base_w, _BPW)], hraw)
    pltpu.sync_copy(t1.at[pl.ds(base_w, _BPW)], traw)
    pltpu.sync_copy(r1i.at[pl.ds(base_w, _BPW)], rraw)
    pltpu.sync_copy(rl1f, rl1res)
    pltpu.sync_copy(rl2f, rl2res)

    lanes = lax.iota(jnp.int32, 16)
    zero16 = jnp.zeros((16,), jnp.float32)

    def build(c, slot):
        base = c * _CH

        def grp(g, carry):
            hv = hraw[pl.ds(base + g * 16, 16)]
            tv = traw[pl.ds(base + g * 16, 16)]

            def jbody(j, carry_j):
                hvj, tvj = carry_j
                row = g * 4 + (j >> 3)
                col = (j & 7) * 16
                hidx[slot, row, pl.ds(col, 16)] = hvj
                tidx[slot, row, pl.ds(col, 16)] = tvj
                return hvj + _E, tvj + _E

            lax.fori_loop(0, _H, jbody, (hv, tv), unroll=4)
            return carry

        lax.fori_loop(0, _CH // 16, grp, 0)

    def fire(c, slot):
        sem = sems.at[slot]

        def lists(k, carry):
            dst = pl.ds(k * 128, 128)
            pltpu.async_copy(e1f.at[hidx.at[slot, k]], be1h.at[slot, dst], sem)
            pltpu.async_copy(e2f.at[hidx.at[slot, k]], be2h.at[slot, dst], sem)
            pltpu.async_copy(e1f.at[tidx.at[slot, k]], be1t.at[slot, dst], sem)
            pltpu.async_copy(e2f.at[tidx.at[slot, k]], be2t.at[slot, dst], sem)
            return carry

        lax.fori_loop(0, _NL, lists, 0)

    def drain(slot):
        pltpu.make_async_copy(
            y_in.at[pl.ds(0, 4 * _CH * _H)],
            drainbuf,
            sems.at[slot],
        ).wait()

    def compute(c, slot, acc):
        base = c * _CH

        def grp(g, acc_g):
            rv = rraw[pl.ds(base + g * 16, 16)]

            def jbody(j, carry):
                res16, acc_j, rvj = carry
                o = g * 512 + j * 16
                a = be1h[slot, pl.ds(o, 16)]
                b = be2h[slot, pl.ds(o, 16)]
                c_ = be1t[slot, pl.ds(o, 16)]
                d = be2t[slot, pl.ds(o, 16)]
                p = plsc.load_gather(rl1res, [rvj])
                q = plsc.load_gather(rl2res, [rvj])
                res16 = res16 + (a * c_ + b * d) * p + (a * d - b * c_) * q
                acc_j = acc_j + a * a + b * b + c_ * c_ + d * d + p * p + q * q
                return res16, acc_j, rvj + _R

            res16, acc_g, _ = lax.fori_loop(0, _H, jbody, (zero16, acc_g, rv),
                                            unroll=2)
            resbuf[pl.ds(base + g * 16, 16)] = res16
            return acc_g

        return lax.fori_loop(0, _CH // 16, grp, acc)

    acc = zero16
    build(0, 0)
    fire(0, 0)
    for c in range(_NCH):
        if c + 1 < _NCH:
            build(c + 1, (c + 1) % 2)
            fire(c + 1, (c + 1) % 2)
        drain(c % 2)
        acc = compute(c, c % 2, acc)

    accbuf[...] = acc
    pltpu.sync_copy(resbuf, res_out.at[wid])
    pltpu.sync_copy(accbuf, regul_out.at[wid])


@jax.jit
def _sc_call(h1, t1, r1i, y_in, e1f, e2f, rl1f, rl2f):
    mesh = plsc.VectorSubcoreMesh(core_axis_name="c", subcore_axis_name="s")
    return pl.kernel(
        _sc_body,
        out_type=[
            jax.ShapeDtypeStruct((_NW, _BPW), jnp.float32),
            jax.ShapeDtypeStruct((_NW, 16), jnp.float32),
        ],
        mesh=mesh,
        compiler_params=pltpu.CompilerParams(needs_layout_passes=False),
        scratch_types=[
            pltpu.VMEM((_BPW,), jnp.int32),
            pltpu.VMEM((_BPW,), jnp.int32),
            pltpu.VMEM((_BPW,), jnp.int32),
            pltpu.VMEM((2, _NL, 128), jnp.int32),
            pltpu.VMEM((2, _NL, 128), jnp.int32),
            pltpu.VMEM((2, _CH * _H), jnp.float32),
            pltpu.VMEM((2, _CH * _H), jnp.float32),
            pltpu.VMEM((2, _CH * _H), jnp.float32),
            pltpu.VMEM((2, _CH * _H), jnp.float32),
            pltpu.VMEM((_R * _H,), jnp.float32),
            pltpu.VMEM((_R * _H,), jnp.float32),
            pltpu.VMEM((4 * _CH * _H,), jnp.float32),
            pltpu.VMEM((_BPW,), jnp.float32),
            pltpu.VMEM((16,), jnp.float32),
            pltpu.SemaphoreType.DMA((2,)),
        ],
    )(h1, t1, r1i, y_in, e1f, e2f, rl1f, rl2f)


def _tc_body(res_ref, y_ref, part_ref, out_ref):
    x = -(y_ref[...] * res_ref[...])
    sp = jnp.maximum(x, 0.0) + jnp.log1p(jnp.exp(-jnp.abs(x)))
    lf = jnp.sum(sp) * (1.0 / _B)
    reg = jnp.sum(part_ref[...]) * (1.0 / (_B * _H))
    out_ref[...] = jnp.reshape(lf + _LMBDA * reg, (1, 1))


def kernel(h, t, r, y, ent1, ent2, rel1, rel2):
    e1f = ent1.T.reshape(_E * _H)
    e2f = ent2.T.reshape(_E * _H)
    rl1f = rel1.T.reshape(_R * _H)
    rl2f = rel2.T.reshape(_R * _H)
    res, parts = _sc_call(h, t, r, y, e1f, e2f, rl1f, rl2f)
    res2 = res.reshape(128, 128)
    y2 = y.reshape(128, 128)
    out = pl.pallas_call(
        _tc_body,
        out_shape=jax.ShapeDtypeStruct((1, 1), jnp.float32),
    )(res2, y2, parts)
    return out[0, 0]


# speedup vs baseline: 5.4175x; 5.4175x over previous
import functools

import jax
import jax.numpy as jnp
from jax import lax
from jax.experimental import pallas as pl
from jax.experimental.pallas import tpu as pltpu
from jax.experimental.pallas import tpu_sc as plsc

_B = 16384
_H = 32
_NW = 32
_BPW = _B // _NW
_CH = 64
_NCH = _BPW // _CH
_LMBDA = 0.0001


def _sc_body(h2, t2, r2, e1, e2, rl1, rl2,
             res_out, regul_out,
             hraw, traw, rraw, hdiv, tdiv, rdiv,
             be1h, be2h, be1t, be2t, br1, br2,
             resbuf, accbuf, sems):
    nc = 2
    wid = lax.axis_index("s") * nc + lax.axis_index("c")

    pltpu.sync_copy(h2.at[wid], hraw)
    pltpu.sync_copy(t2.at[wid], traw)
    pltpu.sync_copy(r2.at[wid], rraw)

    def div_body(i, carry):
        for raw, dv in ((hraw, hdiv), (traw, tdiv), (rraw, rdiv)):
            v = raw[pl.ds(i * 16, 16)]
            dv[i >> 2, pl.ds((i & 3) * 16, 16)] = jnp.right_shift(v, 2)
        return carry

    lax.fori_loop(0, _BPW // 16, div_body, 0)

    def fire(c, slot):
        sem = sems.at[slot]
        return [
            pltpu.async_copy(e1.at[hdiv.at[c]], be1h.at[slot], sem),
            pltpu.async_copy(e2.at[hdiv.at[c]], be2h.at[slot], sem),
            pltpu.async_copy(e1.at[tdiv.at[c]], be1t.at[slot], sem),
            pltpu.async_copy(e2.at[tdiv.at[c]], be2t.at[slot], sem),
            pltpu.async_copy(rl1.at[rdiv.at[c]], br1.at[slot], sem),
            pltpu.async_copy(rl2.at[rdiv.at[c]], br2.at[slot], sem),
        ]

    lanes = lax.iota(jnp.int32, 16)
    zero16 = jnp.zeros((16,), jnp.float32)

    def compute(c, slot, acc):
        base = c * _CH

        def grp(g, acc_g):
            rows = lanes + g * 16
            hm = (hraw[pl.ds(base + g * 16, 16)] & 3) * 32
            tm = (traw[pl.ds(base + g * 16, 16)] & 3) * 32
            rm = (rraw[pl.ds(base + g * 16, 16)] & 3) * 32

            def jbody(j, carry):
                res16, acc_j = carry
                ch = hm + j
                ct = tm + j
                cr = rm + j
                a = plsc.load_gather(be1h.at[slot], [rows, ch])
                b = plsc.load_gather(be2h.at[slot], [rows, ch])
                c_ = plsc.load_gather(be1t.at[slot], [rows, ct])
                d = plsc.load_gather(be2t.at[slot], [rows, ct])
                p = plsc.load_gather(br1.at[slot], [rows, cr])
                q = plsc.load_gather(br2.at[slot], [rows, cr])
                res16 = res16 + (a * c_ + b * d) * p + (a * d - b * c_) * q
                acc_j = acc_j + a * a + b * b + c_ * c_ + d * d + p * p + q * q
                return res16, acc_j

            res16, acc_g = lax.fori_loop(0, _H, jbody, (zero16, acc_g),
                                         unroll=2)
            resbuf[pl.ds(base + g * 16, 16)] = res16
            return acc_g

        return lax.fori_loop(0, _CH // 16, grp, acc)

    acc = zero16
    cps = fire(0, 0)
    for c in range(_NCH):
        nxt = fire(c + 1, (c + 1) % 2) if c + 1 < _NCH else []
        for cp in cps:
            cp.wait()
        acc = compute(c, c % 2, acc)
        cps = nxt

    accbuf[...] = acc
    pltpu.sync_copy(resbuf, res_out.at[wid])
    pltpu.sync_copy(accbuf, regul_out.at[wid])


@jax.jit
def _sc_call(h2, t2, r2, e1, e2, rl1, rl2):
    mesh = plsc.VectorSubcoreMesh(core_axis_name="c", subcore_axis_name="s")
    return pl.kernel(
        _sc_body,
        out_type=[
            jax.ShapeDtypeStruct((_NW, _BPW), jnp.float32),
            jax.ShapeDtypeStruct((_NW, 16), jnp.float32),
        ],
        mesh=mesh,
        compiler_params=pltpu.CompilerParams(needs_layout_passes=False),
        scratch_types=[
            pltpu.VMEM((_BPW,), jnp.int32),
            pltpu.VMEM((_BPW,), jnp.int32),
            pltpu.VMEM((_BPW,), jnp.int32),
            pltpu.VMEM((_NCH, _CH), jnp.int32),
            pltpu.VMEM((_NCH, _CH), jnp.int32),
            pltpu.VMEM((_NCH, _CH), jnp.int32),
            pltpu.VMEM((2, _CH, 128), jnp.float32),
            pltpu.VMEM((2, _CH, 128), jnp.float32),
            pltpu.VMEM((2, _CH, 128), jnp.float32),
            pltpu.VMEM((2, _CH, 128), jnp.float32),
            pltpu.VMEM((2, _CH, 128), jnp.float32),
            pltpu.VMEM((2, _CH, 128), jnp.float32),
            pltpu.VMEM((_BPW,), jnp.float32),
            pltpu.VMEM((16,), jnp.float32),
            pltpu.SemaphoreType.DMA((2,)),
        ],
    )(h2, t2, r2, e1, e2, rl1, rl2)


def _tc_body(res_ref, y_ref, part_ref, out_ref):
    x = -(y_ref[...] * res_ref[...])
    sp = jnp.maximum(x, 0.0) + jnp.log1p(jnp.exp(-jnp.abs(x)))
    lf = jnp.sum(sp) * (1.0 / _B)
    reg = jnp.sum(part_ref[...]) * (1.0 / (_B * _H))
    out_ref[...] = jnp.reshape(lf + _LMBDA * reg, (1, 1))


def kernel(h, t, r, y, ent1, ent2, rel1, rel2):
    h2 = h.reshape(_NW, _BPW)
    t2 = t.reshape(_NW, _BPW)
    r2 = r.reshape(_NW, _BPW)
    e1 = ent1.reshape(-1, 128)
    e2 = ent2.reshape(-1, 128)
    rl1 = rel1.reshape(-1, 128)
    rl2 = rel2.reshape(-1, 128)
    res, parts = _sc_call(h2, t2, r2, e1, e2, rl1, rl2)
    res2 = res.reshape(128, 128)
    y2 = y.reshape(128, 128)
    out = pl.pallas_call(
        _tc_body,
        out_shape=jax.ShapeDtypeStruct((1, 1), jnp.float32),
    )(res2, y2, parts)
    return out[0, 0]


# speedup vs baseline: 5.6919x; 1.0506x over previous
import functools

import jax
import jax.numpy as jnp
from jax import lax
from jax.experimental import pallas as pl
from jax.experimental.pallas import tpu as pltpu
from jax.experimental.pallas import tpu_sc as plsc

_B = 16384
_H = 32
_NW = 32
_BPW = _B // _NW
_NCHUNK = 4
_CHUNK = _BPW // _NCHUNK
_LMBDA = 0.0001


def _sc_body(h_hbm, t_hbm, r_hbm, ent1, ent2, rel1, rel2,
             res_out, regul_out,
             idx_h, idx_t, idx_r,
             e1h, e2h, e1t, e2t, rv1, rv2,
             dbuf, resbuf, accbuf, sem):
    nc = 2
    wid = lax.axis_index("s") * nc + lax.axis_index("c")

    pltpu.sync_copy(h_hbm.at[wid], idx_h)
    pltpu.sync_copy(t_hbm.at[wid], idx_t)
    pltpu.sync_copy(r_hbm.at[wid], idx_r)

    cps = []
    for tbl, idx, dst in ((ent1, idx_h, e1h), (ent2, idx_h, e2h),
                          (ent1, idx_t, e1t), (ent2, idx_t, e2t),
                          (rel1, idx_r, rv1), (rel2, idx_r, rv2)):
        for j in range(_NCHUNK):
            cps.append(pltpu.async_copy(
                tbl.at[idx.at[j]], dst.at[pl.ds(j * _CHUNK, _CHUNK)], sem))
    for cp in cps:
        cp.wait()

    def row_body(b, acc):
        d_off = b * 16
        tot = jnp.zeros((16,), jnp.float32)
        for half in range(2):
            sl = pl.ds(half * 16, 16)
            a = e1h[b, sl]
            bb = e2h[b, sl]
            c = e1t[b, sl]
            d = e2t[b, sl]
            p = rv1[b, sl]
            q = rv2[b, sl]
            tot = tot + (a * c + bb * d) * p + (a * d - bb * c) * q
            acc = acc + a * a + bb * bb + c * c + d * d + p * p + q * q
        dbuf[pl.ds(d_off, 16)] = tot
        return acc

    acc = lax.fori_loop(0, _BPW, row_body,
                        jnp.zeros((16,), jnp.float32), unroll=2)

    lanes16 = lax.iota(jnp.int32, 16) * 16

    def grp_body(g, carry):
        base = g * 256
        acc16 = jnp.zeros((16,), jnp.float32)
        for j in range(16):
            acc16 = acc16 + plsc.load_gather(dbuf, [base + lanes16 + j])
        resbuf[pl.ds(g * 16, 16)] = acc16
        return carry

    lax.fori_loop(0, _BPW // 16, grp_body, 0, unroll=2)

    accbuf[...] = acc
    pltpu.sync_copy(resbuf, res_out.at[wid])
    pltpu.sync_copy(accbuf, regul_out.at[wid])


@functools.partial(jax.jit, static_argnames=())
def _sc_call(h3, t3, r3, ent1, ent2, rel1, rel2):
    mesh = plsc.VectorSubcoreMesh(core_axis_name="c", subcore_axis_name="s")
    return pl.kernel(
        _sc_body,
        out_type=[
            jax.ShapeDtypeStruct((_NW, _BPW), jnp.float32),
            jax.ShapeDtypeStruct((_NW, 16), jnp.float32),
        ],
        mesh=mesh,
        compiler_params=pltpu.CompilerParams(
            needs_layout_passes=False, use_tc_tiling_on_sc=False),
        scratch_types=[
            pltpu.VMEM((_NCHUNK, _CHUNK), jnp.int32),
            pltpu.VMEM((_NCHUNK, _CHUNK), jnp.int32),
            pltpu.VMEM((_NCHUNK, _CHUNK), jnp.int32),
            pltpu.VMEM((_BPW, _H), jnp.float32),
            pltpu.VMEM((_BPW, _H), jnp.float32),
            pltpu.VMEM((_BPW, _H), jnp.float32),
            pltpu.VMEM((_BPW, _H), jnp.float32),
            pltpu.VMEM((_BPW, _H), jnp.float32),
            pltpu.VMEM((_BPW, _H), jnp.float32),
            pltpu.VMEM((_BPW * 16,), jnp.float32),
            pltpu.VMEM((_BPW,), jnp.float32),
            pltpu.VMEM((16,), jnp.float32),
            pltpu.SemaphoreType.DMA,
        ],
    )(h3, t3, r3, ent1, ent2, rel1, rel2)


def _tc_body(res_ref, y_ref, part_ref, out_ref):
    x = -(y_ref[...] * res_ref[...])
    sp = jnp.maximum(x, 0.0) + jnp.log1p(jnp.exp(-jnp.abs(x)))
    lf = jnp.sum(sp) * (1.0 / _B)
    reg = jnp.sum(part_ref[...]) * (1.0 / (_B * _H))
    out_ref[...] = jnp.reshape(lf + _LMBDA * reg, (1, 1))


def kernel(h, t, r, y, ent1, ent2, rel1, rel2):
    h3 = h.reshape(_NW, _NCHUNK, _CHUNK)
    t3 = t.reshape(_NW, _NCHUNK, _CHUNK)
    r3 = r.reshape(_NW, _NCHUNK, _CHUNK)
    res, parts = _sc_call(h3, t3, r3, ent1, ent2, rel1, rel2)
    res2 = res.reshape(128, 128)
    y2 = y.reshape(128, 128)
    out = pl.pallas_call(
        _tc_body,
        out_shape=jax.ShapeDtypeStruct((1, 1), jnp.float32),
    )(res2, y2, parts)
    return out[0, 0]


# speedup vs baseline: 7.7223x; 1.3567x over previous
import jax
import jax.numpy as jnp
from jax import lax
from jax.experimental import pallas as pl
from jax.experimental.pallas import tpu as pltpu
from jax.experimental.pallas import tpu_sc as plsc

_B = 16384
_H = 32
_NW = 32
_BPW = _B // _NW
_CH = 64
_NCH = _BPW // _CH
_LMBDA = 0.0001


def _sc_body(h2, t2, r2, y_in, ent1, ent2, rel1, rel2,
             res_out, regul_out,
             hraw, traw, rraw,
             be1h, be2h, be1t, be2t, br1, br2,
             drainbuf, resbuf, accbuf, sems):
    nc = 2
    wid = lax.axis_index("s") * nc + lax.axis_index("c")

    pltpu.sync_copy(h2.at[wid], hraw)
    pltpu.sync_copy(t2.at[wid], traw)
    pltpu.sync_copy(r2.at[wid], rraw)

    def fire(c, slot):
        base = c * _CH
        sem = sems.at[slot]

        def grp16(g, carry):
            hv = hraw[pl.ds(base + g * 16, 16)]
            tv = traw[pl.ds(base + g * 16, 16)]
            rv = rraw[pl.ds(base + g * 16, 16)]
            for l in range(16):
                b = g * 16 + l
                ih = hv[l]
                it = tv[l]
                ir = rv[l]
                pltpu.async_copy(
                    ent1.at[ih], be1h.at[slot, b, pl.ds(0, _H)], sem)
                pltpu.async_copy(
                    ent2.at[ih], be2h.at[slot, b, pl.ds(0, _H)], sem)
                pltpu.async_copy(
                    ent1.at[it], be1t.at[slot, b, pl.ds(0, _H)], sem)
                pltpu.async_copy(
                    ent2.at[it], be2t.at[slot, b, pl.ds(0, _H)], sem)
                pltpu.async_copy(
                    rel1.at[ir], br1.at[slot, b, pl.ds(0, _H)], sem)
                pltpu.async_copy(
                    rel2.at[ir], br2.at[slot, b, pl.ds(0, _H)], sem)
            return carry

        lax.fori_loop(0, _CH // 16, grp16, 0)

    def drain(slot):
        pltpu.make_async_copy(
            y_in.at[pl.ds(0, 6 * _CH * _H)],
            drainbuf,
            sems.at[slot],
        ).wait()

    lanes = lax.iota(jnp.int32, 16)
    zero16 = jnp.zeros((16,), jnp.float32)
    zero16i = jnp.zeros((16,), jnp.int32)

    def compute(c, slot, acc):
        base = c * _CH

        def grp(g, acc_g):
            rows = lanes + g * 16

            def jbody(j, carry):
                res16, acc_j = carry
                jv = zero16i + j
                a = plsc.load_gather(be1h.at[slot], [rows, jv])
                b = plsc.load_gather(be2h.at[slot], [rows, jv])
                c_ = plsc.load_gather(be1t.at[slot], [rows, jv])
                d = plsc.load_gather(be2t.at[slot], [rows, jv])
                p = plsc.load_gather(br1.at[slot], [rows, jv])
                q = plsc.load_gather(br2.at[slot], [rows, jv])
                res16 = res16 + (a * c_ + b * d) * p + (a * d - b * c_) * q
                acc_j = acc_j + a * a + b * b + c_ * c_ + d * d + p * p + q * q
                return res16, acc_j

            res16, acc_g = lax.fori_loop(0, _H, jbody, (zero16, acc_g),
                                         unroll=2)
            resbuf[pl.ds(base + g * 16, 16)] = res16
            return acc_g

        return lax.fori_loop(0, _CH // 16, grp, acc)

    acc = zero16
    fire(0, 0)
    for c in range(_NCH):
        if c + 1 < _NCH:
            fire(c + 1, (c + 1) % 2)
        drain(c % 2)
        acc = compute(c, c % 2, acc)

    accbuf[...] = acc
    pltpu.sync_copy(resbuf, res_out.at[wid])
    pltpu.sync_copy(accbuf, regul_out.at[wid])


@jax.jit
def _sc_call(h2, t2, r2, y_in, ent1, ent2, rel1, rel2):
    mesh = plsc.VectorSubcoreMesh(core_axis_name="c", subcore_axis_name="s")
    return pl.kernel(
        _sc_body,
        out_type=[
            jax.ShapeDtypeStruct((_NW, _BPW), jnp.float32),
            jax.ShapeDtypeStruct((_NW, 16), jnp.float32),
        ],
        mesh=mesh,
        compiler_params=pltpu.CompilerParams(needs_layout_passes=False),
        scratch_types=[
            pltpu.VMEM((_BPW,), jnp.int32),
            pltpu.VMEM((_BPW,), jnp.int32),
            pltpu.VMEM((_BPW,), jnp.int32),
            pltpu.VMEM((2, _CH, 128), jnp.float32),
            pltpu.VMEM((2, _CH, 128), jnp.float32),
            pltpu.VMEM((2, _CH, 128), jnp.float32),
            pltpu.VMEM((2, _CH, 128), jnp.float32),
            pltpu.VMEM((2, _CH, 128), jnp.float32),
            pltpu.VMEM((2, _CH, 128), jnp.float32),
            pltpu.VMEM((6 * _CH * _H,), jnp.float32),
            pltpu.VMEM((_BPW,), jnp.float32),
            pltpu.VMEM((16,), jnp.float32),
            pltpu.SemaphoreType.DMA((2,)),
        ],
    )(h2, t2, r2, y_in, ent1, ent2, rel1, rel2)


def _tc_body(res_ref, y_ref, part_ref, out_ref):
    x = -(y_ref[...] * res_ref[...])
    sp = jnp.maximum(x, 0.0) + jnp.log1p(jnp.exp(-jnp.abs(x)))
    lf = jnp.sum(sp) * (1.0 / _B)
    reg = jnp.sum(part_ref[...]) * (1.0 / (_B * _H))
    out_ref[...] = jnp.reshape(lf + _LMBDA * reg, (1, 1))


def kernel(h, t, r, y, ent1, ent2, rel1, rel2):
    h2 = h.reshape(_NW, _BPW)
    t2 = t.reshape(_NW, _BPW)
    r2 = r.reshape(_NW, _BPW)
    res, parts = _sc_call(h2, t2, r2, y, ent1, ent2, rel1, rel2)
    res2 = res.reshape(128, 128)
    y2 = y.reshape(128, 128)
    out = pl.pallas_call(
        _tc_body,
        out_shape=jax.ShapeDtypeStruct((1, 1), jnp.float32),
    )(res2, y2, parts)
    return out[0, 0]
